# baseline (speedup 1.0000x reference)
import jax
import jax.numpy as jnp
from jax.experimental import pallas as pl
from jax.experimental.pallas import tpu as pltpu

_DELTA = 1.0
_NCORES = 2
_BLOCK_ROWS = 512
_ROW_CHUNK = 8
_COL_CHUNK = 1024


def _huber_sum_body(yhat_ref, y_ref, o_ref, acc_ref):
    j = pl.program_id(1)
    nsteps = pl.num_programs(1)
    cols = yhat_ref.shape[1]

    def accum(init):
        for r in range(0, _BLOCK_ROWS, _ROW_CHUNK):
            rs = slice(r, r + _ROW_CHUNK)
            for c in range(0, cols, _COL_CHUNK):
                cs = slice(c, c + _COL_CHUNK)
                d = yhat_ref[rs, cs] - y_ref[rs, cs]
                ad = jnp.abs(d)
                m = jnp.minimum(ad, _DELTA)
                h = m * (ad - 0.5 * m)
                if init:
                    acc_ref[rs, cs] = h
                else:
                    acc_ref[rs, cs] += h

    @pl.when(j == 0)
    def _():
        accum(True)

    @pl.when(j > 0)
    def _():
        accum(False)

    @pl.when(j == nsteps - 1)
    def _():
        for c in range(0, cols, _COL_CHUNK):
            cs = slice(c, c + _COL_CHUNK)
            s = acc_ref[0:_ROW_CHUNK, cs]
            for r in range(_ROW_CHUNK, _BLOCK_ROWS, _ROW_CHUNK):
                s = s + acc_ref[r:r + _ROW_CHUNK, cs]
            o_ref[0, :, cs] = s


def kernel(yhat, y):
    rows, cols = yhat.shape
    steps = rows // (_NCORES * _BLOCK_ROWS)
    partials = pl.pallas_call(
        _huber_sum_body,
        out_shape=jax.ShapeDtypeStruct((_NCORES, 8, cols), jnp.float32),
        grid=(_NCORES, steps),
        in_specs=[
            pl.BlockSpec((_BLOCK_ROWS, cols), lambda i, j: (i * steps + j, 0)),
            pl.BlockSpec((_BLOCK_ROWS, cols), lambda i, j: (i * steps + j, 0)),
        ],
        out_specs=pl.BlockSpec((1, 8, cols), lambda i, j: (i, 0, 0)),
        scratch_shapes=[pltpu.VMEM((_BLOCK_ROWS, cols), jnp.float32)],
        compiler_params=pltpu.CompilerParams(
            dimension_semantics=("parallel", "arbitrary"),
            vmem_limit_bytes=56 * 1024 * 1024,
        ),
        name="huber_mean",
    )(yhat, y)
    return jnp.sum(partials) * (1.0 / (rows * cols))


# speedup vs baseline: 1.0055x; 1.0055x over previous
import jax
import jax.numpy as jnp
from jax.experimental import pallas as pl
from jax.experimental.pallas import tpu as pltpu

_DELTA = 1.0
_NCORES = 2
_BLOCK_ROWS = 256
_ROW_CHUNK = 8
_COL_CHUNK = 1024


def _huber_sum_body(yhat_ref, y_ref, o_ref, acc_ref):
    j = pl.program_id(1)
    nsteps = pl.num_programs(1)
    cols = yhat_ref.shape[1]

    def accum(init):
        for r in range(0, _BLOCK_ROWS, _ROW_CHUNK):
            rs = slice(r, r + _ROW_CHUNK)
            for c in range(0, cols, _COL_CHUNK):
                cs = slice(c, c + _COL_CHUNK)
                d = yhat_ref[rs, cs] - y_ref[rs, cs]
                ad = jnp.abs(d)
                m = jnp.minimum(ad, _DELTA)
                h = m * (ad - 0.5 * m)
                if init:
                    acc_ref[rs, cs] = h
                else:
                    acc_ref[rs, cs] += h

    @pl.when(j == 0)
    def _():
        accum(True)

    @pl.when(j > 0)
    def _():
        accum(False)

    @pl.when(j == nsteps - 1)
    def _():
        for c in range(0, cols, _COL_CHUNK):
            cs = slice(c, c + _COL_CHUNK)
            s = acc_ref[0:_ROW_CHUNK, cs]
            for r in range(_ROW_CHUNK, _BLOCK_ROWS, _ROW_CHUNK):
                s = s + acc_ref[r:r + _ROW_CHUNK, cs]
            o_ref[0, :, cs] = s


def kernel(yhat, y):
    rows, cols = yhat.shape
    steps = rows // (_NCORES * _BLOCK_ROWS)
    partials = pl.pallas_call(
        _huber_sum_body,
        out_shape=jax.ShapeDtypeStruct((_NCORES, 8, cols), jnp.float32),
        grid=(_NCORES, steps),
        in_specs=[
            pl.BlockSpec((_BLOCK_ROWS, cols), lambda i, j: (i * steps + j, 0)),
            pl.BlockSpec((_BLOCK_ROWS, cols), lambda i, j: (i * steps + j, 0)),
        ],
        out_specs=pl.BlockSpec((1, 8, cols), lambda i, j: (i, 0, 0)),
        scratch_shapes=[pltpu.VMEM((_BLOCK_ROWS, cols), jnp.float32)],
        compiler_params=pltpu.CompilerParams(
            dimension_semantics=("parallel", "arbitrary"),
            vmem_limit_bytes=56 * 1024 * 1024,
        ),
        name="huber_mean",
    )(yhat, y)
    return jnp.sum(partials) * (1.0 / (rows * cols))


# speedup vs baseline: 1.0104x; 1.0049x over previous
import jax
import jax.numpy as jnp
from jax.experimental import pallas as pl
from jax.experimental.pallas import tpu as pltpu

_DELTA = 1.0
_NCORES = 2
_BLOCK_ROWS = 256
_ROW_CHUNK = 8
_COL_CHUNK = 1024


def _huber_sum_body(yhat_ref, y_ref, o_ref, acc_ref):
    j = pl.program_id(1)
    nsteps = pl.num_programs(1)
    cols = yhat_ref.shape[1]

    def accum(init):
        for r in range(0, _BLOCK_ROWS, _ROW_CHUNK):
            rs = slice(r, r + _ROW_CHUNK)
            for c in range(0, cols, _COL_CHUNK):
                cs = slice(c, c + _COL_CHUNK)
                d = yhat_ref[rs, cs] - y_ref[rs, cs]
                ad = jnp.abs(d)
                m = jnp.minimum(ad, _DELTA)
                h = m * (ad - 0.5 * m)
                if init:
                    acc_ref[rs, cs] = h
                else:
                    acc_ref[rs, cs] += h

    @pl.when(j == 0)
    def _():
        accum(True)

    @pl.when(j > 0)
    def _():
        accum(False)

    @pl.when(j == nsteps - 1)
    def _():
        for c in range(0, cols, _COL_CHUNK):
            cs = slice(c, c + _COL_CHUNK)
            s = acc_ref[0:_ROW_CHUNK, cs]
            for r in range(_ROW_CHUNK, _BLOCK_ROWS, _ROW_CHUNK):
                s = s + acc_ref[r:r + _ROW_CHUNK, cs]
            o_ref[0, :, cs] = s


def kernel(yhat, y):
    rows, cols = yhat.shape
    steps = rows // (_NCORES * _BLOCK_ROWS)
    partials = pl.pallas_call(
        _huber_sum_body,
        out_shape=jax.ShapeDtypeStruct((_NCORES, 8, cols), jnp.float32),
        grid=(_NCORES, steps),
        in_specs=[
            pl.BlockSpec((_BLOCK_ROWS, cols), lambda i, j: (j * _NCORES + i, 0)),
            pl.BlockSpec((_BLOCK_ROWS, cols), lambda i, j: (j * _NCORES + i, 0)),
        ],
        out_specs=pl.BlockSpec((1, 8, cols), lambda i, j: (i, 0, 0)),
        scratch_shapes=[pltpu.VMEM((_BLOCK_ROWS, cols), jnp.float32)],
        compiler_params=pltpu.CompilerParams(
            dimension_semantics=("parallel", "arbitrary"),
            vmem_limit_bytes=56 * 1024 * 1024,
        ),
        name="huber_mean",
    )(yhat, y)
    return jnp.sum(partials) * (1.0 / (rows * cols))
